# baseline (speedup 1.0000x reference)
import functools
import math

import jax
import jax.numpy as jnp
from jax import lax
from jax.experimental import pallas as pl
from jax.experimental.pallas import tpu as pltpu
from jax.experimental.pallas import tpu_sc as plsc

_K = 21
_KP = 32
_INL = 6.0
_NEG = -3.0e38
_C0 = math.exp(-6.0)
_NW = 32
_F = 64
_FS = 64


def _emb_body(x_ref, w1_ref, w2_ref, emb_ref):
    h = jnp.tanh(x_ref[...] * w1_ref[...]) * w2_ref[...]
    nrm = jnp.sqrt(jnp.sum(h * h, axis=1, keepdims=True))
    emb_ref[...] = h / jnp.maximum(nrm, 1e-12)


def _simtopk_body(emb_blk_ref, emb_all_ref, gv_ref, ix_ref, rsg_ref,
                  csum_ref):
    i = pl.program_id(0)
    a = emb_blk_ref[...]
    b = emb_all_ref[...]
    s0 = jax.lax.dot_general(a, b, (((1,), (1,)), ((), ())),
                             preferred_element_type=jnp.float32)
    n = s0.shape[1]
    br = s0.shape[0]
    col = jax.lax.broadcasted_iota(jnp.int32, s0.shape, 1)
    imask = n - 1
    bits = jax.lax.bitcast_convert_type(s0, jnp.int32)
    p = jax.lax.bitcast_convert_type((bits & ~imask) | (imask - col),
                                     jnp.float32)
    vals = []
    idxs = []
    for _ in range(_K):
        m = jnp.max(p, axis=1, keepdims=True)
        p = jnp.where(p == m, _NEG, p)
        mb = jax.lax.bitcast_convert_type(m, jnp.int32)
        vals.append(jax.lax.bitcast_convert_type(mb & ~imask, jnp.float32))
        idxs.append(imask - (mb & imask))
    t = jnp.concatenate(vals, axis=1)
    ix = jnp.concatenate(idxs, axis=1)
    z = t * _INL - _INL
    g = jnp.where(z > 0, z + 1.0, jnp.exp(z)) - _C0
    rsg_ref[...] = jnp.sum(g, axis=1, keepdims=True)
    pad = jnp.zeros((br, _KP - _K), jnp.float32)
    ipad = jnp.zeros((br, _KP - _K), jnp.int32)
    gv_ref[...] = jnp.concatenate([g, pad], axis=1)
    ix_ref[...] = jnp.concatenate([ix, ipad], axis=1)
    s0t = jax.lax.bitcast_convert_type(bits & ~imask, jnp.float32)
    zf = jnp.where(p == _NEG, s0t, 0.0) * _INL - _INL
    gblk = jnp.where(p == _NEG,
                     jnp.where(zf > 0, zf + 1.0, jnp.exp(zf)) - _C0, 0.0)
    onesv = jnp.full((br, 1), 1.0, jnp.float32)
    cs = jax.lax.dot_general(gblk, onesv, (((0,), (0,)), ((), ())),
                             preferred_element_type=jnp.float32)

    @pl.when(i == 0)
    def _():
        csum_ref[...] = jnp.zeros_like(csum_ref)

    csum_ref[...] += cs


def _spmm_body(ix_hbm, gv_hbm, mu_hbm, h_out_hbm,
               idx_v, g_v, mu_loc, gacc, buf, bufs, h_sh, own_idx, sem):
    cid = lax.axis_index("c")
    sid = lax.axis_index("s")
    wid = sid * 2 + cid
    n = mu_hbm.shape[0]
    rows = n // _NW
    nchunks = idx_v.shape[0]
    crows = rows // nchunks
    tile_rows = n // 16
    base = wid * rows

    pltpu.async_copy(ix_hbm.at[wid], idx_v, sem).wait()
    pltpu.async_copy(gv_hbm.at[wid], g_v, sem).wait()
    pltpu.async_copy(mu_hbm.at[pl.ds(base, rows)], mu_loc, sem).wait()

    def zg(r, c):
        for q in range(_F // 16):
            gacc[r, pl.ds(q * 16, 16)] = jnp.zeros((16,), jnp.float32)
        return c

    lax.fori_loop(0, rows, zg, 0)

    lanes = lax.iota(jnp.int32, 16)

    def zi(i, c):
        j = i // 8
        own_idx[j, pl.ds((i % 8) * 16, 16)] = base + i * 16 + lanes
        return c

    lax.fori_loop(0, rows // 16, zi, 0)

    def zb(r, c):
        for q in range(_F // 16):
            bufs[r, pl.ds(q * 16, 16)] = jnp.zeros((16,), jnp.float32)
        return c

    lax.fori_loop(0, bufs.shape[0], zb, 0)

    def zh(j, c):
        pltpu.sync_copy(bufs,
                        h_sh.at[pl.ds(sid * tile_rows + j * bufs.shape[0],
                                      bufs.shape[0])])
        return c

    lax.fori_loop(0, tile_rows // bufs.shape[0], zh, 0)
    plsc.subcore_barrier()

    def chunk_body(cc, c):
        idxrow = idx_v.at[cc]
        pltpu.async_copy(mu_hbm.at[idxrow], buf, sem).wait()

        def row_gs(r4, c2):
            rr = cc * crows + r4
            gv1 = g_v[cc, pl.ds(r4 * _KP, 16)]
            gv2 = g_v[cc, pl.ds(r4 * _KP + 16, 16)]
            ml = [mu_loc[rr, pl.ds(q * 16, 16)] for q in range(_F // 16)]
            for k in range(_KP):
                gsc = gv1[k] if k < 16 else gv2[k - 16]
                e = r4 * _KP + k
                for q in range(_F // 16):
                    gb = buf[e, pl.ds(q * 16, 16)]
                    gacc[rr, pl.ds(q * 16, 16)] = (
                        gacc[rr, pl.ds(q * 16, 16)] + gsc * gb)
                    bufs[e, pl.ds(q * 16, 16)] = gsc * ml[q]
            return c2

        lax.fori_loop(0, crows, row_gs, 0)
        pltpu.sync_copy(bufs, h_sh.at[idxrow], add=True)
        return c

    lax.fori_loop(0, nchunks, chunk_body, 0)
    for j in range(2):
        pltpu.sync_copy(gacc.at[pl.ds(j * 128, 128)],
                        h_sh.at[own_idx.at[j]], add=True)
    plsc.subcore_barrier()

    def dr(j, c):
        sl = pl.ds(sid * tile_rows + j * bufs.shape[0], bufs.shape[0])
        pltpu.sync_copy(h_sh.at[sl], bufs)
        pltpu.sync_copy(bufs, h_out_hbm.at[cid, sl])
        return c

    lax.fori_loop(0, tile_rows // bufs.shape[0], dr, 0)


def _prep_body(x_ref, wg1_ref, bg1_ref, rsg_ref, csp_ref, u_ref, mu1_ref,
               s1_ref):
    n = x_ref.shape[0]
    deg = _C0 * n + 0.5 * (rsg_ref[...] + csp_ref[...])
    u = 1.0 / (jnp.sqrt(deg) + 1e-10)
    u_ref[...] = u
    m1 = jnp.dot(x_ref[...], wg1_ref[...],
                 preferred_element_type=jnp.float32) + bg1_ref[...]
    mu1 = u * m1
    if _FS > mu1.shape[1]:
        mu1_ref[...] = jnp.concatenate(
            [mu1, jnp.zeros((mu1.shape[0], _FS - mu1.shape[1]), jnp.float32)],
            axis=1)
    else:
        mu1_ref[...] = mu1
    s1_ref[...] = jnp.sum(mu1, axis=0, keepdims=True)


def _mid_body(h_ref, u_ref, s1_ref, wg2_ref, bg2_ref, mu2_ref,
              s2_ref):
    gh = h_ref[0] + h_ref[1]
    t = _C0 * s1_ref[...] + 0.5 * gh
    h1 = jnp.maximum(u_ref[...] * t, 0.0)
    m2 = jnp.dot(h1, wg2_ref[...],
                 preferred_element_type=jnp.float32) + bg2_ref[...]
    mu2 = u_ref[...] * m2
    if _FS > mu2.shape[1]:
        mu2_ref[...] = jnp.concatenate(
            [mu2, jnp.zeros((mu2.shape[0], _FS - mu2.shape[1]), jnp.float32)],
            axis=1)
    else:
        mu2_ref[...] = mu2
    s2_ref[...] = jnp.sum(mu2, axis=0, keepdims=True)


def _fin_body(h_ref, u_ref, s2_ref, out_ref):
    gh = h_ref[0] + h_ref[1]
    t = _C0 * s2_ref[...] + 0.5 * gh
    out_ref[...] = u_ref[...] * t


def _make_spmm(n):
    rows = n // _NW
    nchunks = rows * _KP // 128
    mesh = plsc.VectorSubcoreMesh(core_axis_name="c", subcore_axis_name="s")
    return pl.kernel(
        _spmm_body,
        mesh=mesh,
        compiler_params=pltpu.CompilerParams(use_tc_tiling_on_sc=False),
        out_type=jax.ShapeDtypeStruct((2, n, _F), jnp.float32),
        scratch_types=[
            pltpu.VMEM((nchunks, 128), jnp.int32),
            pltpu.VMEM((nchunks, 128), jnp.float32),
            pltpu.VMEM((rows, _FS), jnp.float32),
            pltpu.VMEM((rows, _F), jnp.float32),
            pltpu.VMEM((128, _FS), jnp.float32),
            pltpu.VMEM((128, _F), jnp.float32),
            pltpu.VMEM_SHARED((n, _F), jnp.float32),
            pltpu.VMEM((2, 128), jnp.int32),
            pltpu.SemaphoreType.DMA,
        ],
    )


def kernel(x, w1, w2, Wg1, bg1, Wg2, bg2):
    n, d = x.shape
    hid = Wg1.shape[1]
    outd = Wg2.shape[1]
    br = 256 if n % 256 == 0 else n
    g = n // br

    emb = pl.pallas_call(
        _emb_body,
        out_shape=jax.ShapeDtypeStruct((n, d), jnp.float32),
    )(x, w1.reshape(1, d), w2.reshape(1, d))

    gv, ix, rsg, csp = pl.pallas_call(
        _simtopk_body,
        grid=(g,),
        in_specs=[pl.BlockSpec((br, d), lambda i: (i, 0)),
                  pl.BlockSpec((n, d), lambda i: (0, 0))],
        out_specs=[pl.BlockSpec((br, _KP), lambda i: (i, 0)),
                   pl.BlockSpec((br, _KP), lambda i: (i, 0)),
                   pl.BlockSpec((br, 1), lambda i: (i, 0)),
                   pl.BlockSpec((n, 1), lambda i: (0, 0))],
        out_shape=[jax.ShapeDtypeStruct((n, _KP), jnp.float32),
                   jax.ShapeDtypeStruct((n, _KP), jnp.int32),
                   jax.ShapeDtypeStruct((n, 1), jnp.float32),
                   jax.ShapeDtypeStruct((n, 1), jnp.float32)],
    )(emb, emb)

    rows = n // _NW
    nchunks = rows * _KP // 128

    u, mu1, s1 = pl.pallas_call(
        _prep_body,
        out_shape=[jax.ShapeDtypeStruct((n, 1), jnp.float32),
                   jax.ShapeDtypeStruct((n, _FS), jnp.float32),
                   jax.ShapeDtypeStruct((1, hid), jnp.float32)],
    )(x, Wg1, bg1.reshape(1, hid), rsg, csp)

    ixr = ix.reshape(_NW, nchunks, 128)
    gvr = gv.reshape(_NW, nchunks, 128)
    spmm = _make_spmm(n)

    h1p = spmm(ixr, gvr, mu1)

    mu2, s2 = pl.pallas_call(
        _mid_body,
        out_shape=[jax.ShapeDtypeStruct((n, _FS), jnp.float32),
                   jax.ShapeDtypeStruct((1, outd), jnp.float32)],
    )(h1p, u, s1, Wg2, bg2.reshape(1, outd))

    h2p = spmm(ixr, gvr, mu2)

    out = pl.pallas_call(
        _fin_body,
        out_shape=jax.ShapeDtypeStruct((n, outd), jnp.float32),
    )(h2p, u, s2)

    return out


# speedup vs baseline: 1.0008x; 1.0008x over previous
import functools
import math

import jax
import jax.numpy as jnp
from jax import lax
from jax.experimental import pallas as pl
from jax.experimental.pallas import tpu as pltpu
from jax.experimental.pallas import tpu_sc as plsc

_K = 21
_KP = 32
_INL = 6.0
_NEG = -3.0e38
_C0 = math.exp(-6.0)
_NW = 32
_F = 64
_FS = 64


def _emb_body(x_ref, w1_ref, w2_ref, emb_ref):
    h = jnp.tanh(x_ref[...] * w1_ref[...]) * w2_ref[...]
    nrm = jnp.sqrt(jnp.sum(h * h, axis=1, keepdims=True))
    emb_ref[...] = h / jnp.maximum(nrm, 1e-12)


def _simtopk_body(emb_blk_ref, emb_all_ref, gv_ref, ix_ref, rsg_ref,
                  csum_ref):
    i = pl.program_id(0)
    a = emb_blk_ref[...]
    b = emb_all_ref[...]
    s0 = jax.lax.dot_general(a, b, (((1,), (1,)), ((), ())),
                             preferred_element_type=jnp.float32)
    n = s0.shape[1]
    br = s0.shape[0]
    col = jax.lax.broadcasted_iota(jnp.int32, s0.shape, 1)
    imask = n - 1
    bits = jax.lax.bitcast_convert_type(s0, jnp.int32)
    p = jax.lax.bitcast_convert_type((bits & ~imask) | (imask - col),
                                     jnp.float32)
    vals = []
    idxs = []
    for _ in range(_K):
        m = jnp.max(p, axis=1, keepdims=True)
        p = jnp.where(p == m, _NEG, p)
        mb = jax.lax.bitcast_convert_type(m, jnp.int32)
        vals.append(jax.lax.bitcast_convert_type(mb & ~imask, jnp.float32))
        idxs.append(imask - (mb & imask))
    t = jnp.concatenate(vals, axis=1)
    ix = jnp.concatenate(idxs, axis=1)
    z = t * _INL - _INL
    g = jnp.where(z > 0, z + 1.0, jnp.exp(z)) - _C0
    rsg_ref[...] = jnp.sum(g, axis=1, keepdims=True)
    pad = jnp.zeros((br, _KP - _K), jnp.float32)
    ipad = jnp.zeros((br, _KP - _K), jnp.int32)
    gv_ref[...] = jnp.concatenate([g, pad], axis=1)
    ix_ref[...] = jnp.concatenate([ix, ipad], axis=1)
    s0t = jax.lax.bitcast_convert_type(bits & ~imask, jnp.float32)
    zf = jnp.where(p == _NEG, s0t, 0.0) * _INL - _INL
    gblk = jnp.where(p == _NEG,
                     jnp.where(zf > 0, zf + 1.0, jnp.exp(zf)) - _C0, 0.0)
    onesv = jnp.full((br, 1), 1.0, jnp.float32)
    cs = jax.lax.dot_general(gblk, onesv, (((0,), (0,)), ((), ())),
                             preferred_element_type=jnp.float32)

    @pl.when(i == 0)
    def _():
        csum_ref[...] = jnp.zeros_like(csum_ref)

    csum_ref[...] += cs


def _spmm_body(ix_hbm, gv_hbm, mu_hbm, h_out_hbm,
               idx_v, g_v, mu_loc, gacc, buf, bufs, h_sh, own_idx, sem):
    cid = lax.axis_index("c")
    sid = lax.axis_index("s")
    wid = sid * 2 + cid
    n = mu_hbm.shape[0]
    rows = n // _NW
    nchunks = idx_v.shape[0]
    crows = rows // nchunks
    tile_rows = n // 16
    base = wid * rows

    pltpu.async_copy(ix_hbm.at[wid], idx_v, sem).wait()
    pltpu.async_copy(gv_hbm.at[wid], g_v, sem).wait()
    pltpu.async_copy(mu_hbm.at[pl.ds(base, rows)], mu_loc, sem).wait()

    def zg(r, c):
        for q in range(_F // 16):
            gacc[r, pl.ds(q * 16, 16)] = jnp.zeros((16,), jnp.float32)
        return c

    lax.fori_loop(0, rows, zg, 0)

    lanes = lax.iota(jnp.int32, 16)

    def zi(i, c):
        j = i // 8
        own_idx[j, pl.ds((i % 8) * 16, 16)] = base + i * 16 + lanes
        return c

    lax.fori_loop(0, rows // 16, zi, 0)

    def zb(r, c):
        for q in range(_F // 16):
            bufs[r, pl.ds(q * 16, 16)] = jnp.zeros((16,), jnp.float32)
        return c

    lax.fori_loop(0, bufs.shape[0], zb, 0)

    def zh(j, c):
        pltpu.sync_copy(bufs,
                        h_sh.at[pl.ds(sid * tile_rows + j * bufs.shape[0],
                                      bufs.shape[0])])
        return c

    lax.fori_loop(0, tile_rows // bufs.shape[0], zh, 0)
    plsc.subcore_barrier()

    def chunk_body(cc, c):
        idxrow = idx_v.at[cc]
        pltpu.async_copy(mu_hbm.at[idxrow], buf, sem).wait()

        def row_gs(r4, c2):
            rr = cc * crows + r4
            gv1 = g_v[cc, pl.ds(r4 * _KP, 16)]
            gv2 = g_v[cc, pl.ds(r4 * _KP + 16, 16)]
            ml = [mu_loc[rr, pl.ds(q * 16, 16)] for q in range(_F // 16)]
            accs = [gacc[rr, pl.ds(q * 16, 16)] for q in range(_F // 16)]
            for k in range(_KP):
                gsc = gv1[k] if k < 16 else gv2[k - 16]
                e = r4 * _KP + k
                for q in range(_F // 16):
                    accs[q] = accs[q] + gsc * buf[e, pl.ds(q * 16, 16)]
                    bufs[e, pl.ds(q * 16, 16)] = gsc * ml[q]
            for q in range(_F // 16):
                gacc[rr, pl.ds(q * 16, 16)] = accs[q]
            return c2

        lax.fori_loop(0, crows, row_gs, 0)
        pltpu.sync_copy(bufs, h_sh.at[idxrow], add=True)
        return c

    lax.fori_loop(0, nchunks, chunk_body, 0)
    for j in range(2):
        pltpu.sync_copy(gacc.at[pl.ds(j * 128, 128)],
                        h_sh.at[own_idx.at[j]], add=True)
    plsc.subcore_barrier()

    def dr(j, c):
        sl = pl.ds(sid * tile_rows + j * bufs.shape[0], bufs.shape[0])
        pltpu.sync_copy(h_sh.at[sl], bufs)
        pltpu.sync_copy(bufs, h_out_hbm.at[cid, sl])
        return c

    lax.fori_loop(0, tile_rows // bufs.shape[0], dr, 0)


def _prep_body(x_ref, wg1_ref, bg1_ref, rsg_ref, csp_ref, u_ref, mu1_ref,
               s1_ref):
    n = x_ref.shape[0]
    deg = _C0 * n + 0.5 * (rsg_ref[...] + csp_ref[...])
    u = 1.0 / (jnp.sqrt(deg) + 1e-10)
    u_ref[...] = u
    m1 = jnp.dot(x_ref[...], wg1_ref[...],
                 preferred_element_type=jnp.float32) + bg1_ref[...]
    mu1 = u * m1
    if _FS > mu1.shape[1]:
        mu1_ref[...] = jnp.concatenate(
            [mu1, jnp.zeros((mu1.shape[0], _FS - mu1.shape[1]), jnp.float32)],
            axis=1)
    else:
        mu1_ref[...] = mu1
    s1_ref[...] = jnp.sum(mu1, axis=0, keepdims=True)


def _mid_body(h_ref, u_ref, s1_ref, wg2_ref, bg2_ref, mu2_ref,
              s2_ref):
    gh = h_ref[0] + h_ref[1]
    t = _C0 * s1_ref[...] + 0.5 * gh
    h1 = jnp.maximum(u_ref[...] * t, 0.0)
    m2 = jnp.dot(h1, wg2_ref[...],
                 preferred_element_type=jnp.float32) + bg2_ref[...]
    mu2 = u_ref[...] * m2
    if _FS > mu2.shape[1]:
        mu2_ref[...] = jnp.concatenate(
            [mu2, jnp.zeros((mu2.shape[0], _FS - mu2.shape[1]), jnp.float32)],
            axis=1)
    else:
        mu2_ref[...] = mu2
    s2_ref[...] = jnp.sum(mu2, axis=0, keepdims=True)


def _fin_body(h_ref, u_ref, s2_ref, out_ref):
    gh = h_ref[0] + h_ref[1]
    t = _C0 * s2_ref[...] + 0.5 * gh
    out_ref[...] = u_ref[...] * t


def _make_spmm(n):
    rows = n // _NW
    nchunks = rows * _KP // 128
    mesh = plsc.VectorSubcoreMesh(core_axis_name="c", subcore_axis_name="s")
    return pl.kernel(
        _spmm_body,
        mesh=mesh,
        compiler_params=pltpu.CompilerParams(use_tc_tiling_on_sc=False),
        out_type=jax.ShapeDtypeStruct((2, n, _F), jnp.float32),
        scratch_types=[
            pltpu.VMEM((nchunks, 128), jnp.int32),
            pltpu.VMEM((nchunks, 128), jnp.float32),
            pltpu.VMEM((rows, _FS), jnp.float32),
            pltpu.VMEM((rows, _F), jnp.float32),
            pltpu.VMEM((128, _FS), jnp.float32),
            pltpu.VMEM((128, _F), jnp.float32),
            pltpu.VMEM_SHARED((n, _F), jnp.float32),
            pltpu.VMEM((2, 128), jnp.int32),
            pltpu.SemaphoreType.DMA,
        ],
    )


def kernel(x, w1, w2, Wg1, bg1, Wg2, bg2):
    n, d = x.shape
    hid = Wg1.shape[1]
    outd = Wg2.shape[1]
    br = 256 if n % 256 == 0 else n
    g = n // br

    emb = pl.pallas_call(
        _emb_body,
        out_shape=jax.ShapeDtypeStruct((n, d), jnp.float32),
    )(x, w1.reshape(1, d), w2.reshape(1, d))

    gv, ix, rsg, csp = pl.pallas_call(
        _simtopk_body,
        grid=(g,),
        in_specs=[pl.BlockSpec((br, d), lambda i: (i, 0)),
                  pl.BlockSpec((n, d), lambda i: (0, 0))],
        out_specs=[pl.BlockSpec((br, _KP), lambda i: (i, 0)),
                   pl.BlockSpec((br, _KP), lambda i: (i, 0)),
                   pl.BlockSpec((br, 1), lambda i: (i, 0)),
                   pl.BlockSpec((n, 1), lambda i: (0, 0))],
        out_shape=[jax.ShapeDtypeStruct((n, _KP), jnp.float32),
                   jax.ShapeDtypeStruct((n, _KP), jnp.int32),
                   jax.ShapeDtypeStruct((n, 1), jnp.float32),
                   jax.ShapeDtypeStruct((n, 1), jnp.float32)],
    )(emb, emb)

    rows = n // _NW
    nchunks = rows * _KP // 128

    u, mu1, s1 = pl.pallas_call(
        _prep_body,
        out_shape=[jax.ShapeDtypeStruct((n, 1), jnp.float32),
                   jax.ShapeDtypeStruct((n, _FS), jnp.float32),
                   jax.ShapeDtypeStruct((1, hid), jnp.float32)],
    )(x, Wg1, bg1.reshape(1, hid), rsg, csp)

    ixr = ix.reshape(_NW, nchunks, 128)
    gvr = gv.reshape(_NW, nchunks, 128)
    spmm = _make_spmm(n)

    h1p = spmm(ixr, gvr, mu1)

    mu2, s2 = pl.pallas_call(
        _mid_body,
        out_shape=[jax.ShapeDtypeStruct((n, _FS), jnp.float32),
                   jax.ShapeDtypeStruct((1, outd), jnp.float32)],
    )(h1p, u, s1, Wg2, bg2.reshape(1, outd))

    h2p = spmm(ixr, gvr, mu2)

    out = pl.pallas_call(
        _fin_body,
        out_shape=jax.ShapeDtypeStruct((n, outd), jnp.float32),
    )(h2p, u, s2)

    return out
